# Initial kernel scaffold; baseline (speedup 1.0000x reference)
import jax
import jax.numpy as jnp
from jax.experimental import pallas as pl


def kernel(x, dist):
    raise NotImplementedError("write your pallas kernel here")


# speedup vs baseline: 13.6558x; 13.6558x over previous
import functools

import jax
import jax.numpy as jnp
from jax.experimental import pallas as pl
from jax.experimental.pallas import tpu as pltpu

_M0 = 0.05
_NITER = 18


def _dtm_body(w_ref, dist_ref, o_ref):
    w = w_ref[0, 0, :][None, :]
    d2 = dist_ref[...]
    d2 = d2 * d2
    bound = _M0 * jnp.sum(w)
    hi = jnp.max(d2, axis=1, keepdims=True)
    lo = jnp.full_like(hi, -1.0)

    def body(_, carry):
        lo_, hi_ = carry
        mid = 0.5 * (lo_ + hi_)
        wm = jnp.sum(jnp.where(d2 <= mid, w, 0.0), axis=1, keepdims=True)
        ge = wm >= bound
        return jnp.where(ge, lo_, mid), jnp.where(ge, mid, hi_)

    lo, hi = jax.lax.fori_loop(0, _NITER, body, (lo, hi))
    big = jnp.float32(3.0e38)
    t2 = jnp.min(jnp.where(d2 > lo, d2, big), axis=1, keepdims=True)
    t2 = jnp.where(t2 >= big, hi, t2)
    below = d2 < t2
    wb = jnp.sum(jnp.where(below, w, 0.0), axis=1, keepdims=True)
    sb = jnp.sum(jnp.where(below, w * d2, 0.0), axis=1, keepdims=True)
    val = sb + t2 * (bound - wb)
    o_ref[...] = jnp.sqrt(val / bound).reshape(o_ref.shape)


def _dtm(weight, dist, hw):
    bc = weight.shape[0]
    return pl.pallas_call(
        _dtm_body,
        grid=(bc,),
        in_specs=[
            pl.BlockSpec((1, 1, hw), lambda i: (i, 0, 0)),
            pl.BlockSpec((hw, hw), lambda i: (0, 0)),
        ],
        out_specs=pl.BlockSpec((1, 1, hw), lambda i: (i, 0, 0)),
        out_shape=jax.ShapeDtypeStruct((bc, 1, hw), jnp.float32),
    )(weight.reshape(bc, 1, hw), dist)


@jax.jit
def kernel(x, dist):
    B, C, H, W = x.shape
    HW = H * W
    weight = x.reshape(B * C, HW)
    out = _dtm(weight, dist, HW)
    return out.reshape(B, C, H, W)


# speedup vs baseline: 18.1621x; 1.3300x over previous
import functools

import jax
import jax.numpy as jnp
from jax.experimental import pallas as pl
from jax.experimental.pallas import tpu as pltpu

_M0 = 0.05
_NITER = 13
_LO0 = -0.0078125


def _dtm_body(w_ref, dist_ref, o_ref, d2_ref):
    @pl.when(pl.program_id(0) == 0)
    def _():
        d = dist_ref[...]
        d2_ref[...] = d * d

    w = w_ref[0, 0, :][None, :]
    d2 = d2_ref[...]
    bound = _M0 * jnp.sum(w)
    hi = jnp.max(d2, axis=1, keepdims=True)
    lo = jnp.full_like(hi, _LO0)

    def body(_, carry):
        lo_, hi_ = carry
        mid = 0.5 * (lo_ + hi_)
        wm = jnp.sum(jnp.where(d2 <= mid, w, 0.0), axis=1, keepdims=True)
        ge = wm >= bound
        return jnp.where(ge, lo_, mid), jnp.where(ge, mid, hi_)

    lo, hi = jax.lax.fori_loop(0, _NITER, body, (lo, hi))
    big = jnp.float32(3.0e38)
    t2 = jnp.min(jnp.where(d2 > lo, d2, big), axis=1, keepdims=True)
    t2 = jnp.where(t2 >= big, hi, t2)
    below = d2 < t2
    wb = jnp.sum(jnp.where(below, w, 0.0), axis=1, keepdims=True)
    sb = jnp.sum(jnp.where(below, w * d2, 0.0), axis=1, keepdims=True)
    val = sb + t2 * (bound - wb)
    o_ref[...] = jnp.sqrt(val / bound).reshape(o_ref.shape)


def _dtm(weight, dist, hw):
    bc = weight.shape[0]
    return pl.pallas_call(
        _dtm_body,
        grid=(bc,),
        in_specs=[
            pl.BlockSpec((1, 1, hw), lambda i: (i, 0, 0)),
            pl.BlockSpec((hw, hw), lambda i: (0, 0)),
        ],
        out_specs=pl.BlockSpec((1, 1, hw), lambda i: (i, 0, 0)),
        out_shape=jax.ShapeDtypeStruct((bc, 1, hw), jnp.float32),
        scratch_shapes=[pltpu.VMEM((hw, hw), jnp.float32)],
    )(weight.reshape(bc, 1, hw), dist)


@jax.jit
def kernel(x, dist):
    B, C, H, W = x.shape
    HW = H * W
    weight = x.reshape(B * C, HW)
    out = _dtm(weight, dist, HW)
    return out.reshape(B, C, H, W)
